# baseline (speedup 1.0000x reference)
import functools

import jax
import jax.numpy as jnp
from jax import lax
from jax.experimental import pallas as pl
from jax.experimental.pallas import tpu as pltpu
from jax.experimental.pallas import tpu_sc as plsc

_N = 1000
_D = 1024
_R = 16
_TOPK = 50
_KPAD = 64
_BETA2 = 0.04
_TAU = 0.01
_SCALING = 2.0
_KSMALL = 4


def _routing_body(q_ref, c_ref, w_ref, ria_ref, cnt_ref, idx_ref):
    q = q_ref[...]
    C = c_ref[...]
    qnorm = jnp.sqrt(jnp.sum(q * q)) + 1e-9
    ones = jnp.ones((1, _D), jnp.float32)
    dn = (((1,), (1,)), ((), ()))
    dots = lax.dot_general(q, C, dn, preferred_element_type=jnp.float32)
    cn2 = lax.dot_general(ones, C * C, dn, preferred_element_type=jnp.float32)
    cnorm = jnp.sqrt(cn2) + 1e-9
    sim = dots / (qnorm * cnorm * _BETA2)
    m0 = jnp.max(sim)
    e = jnp.exp(sim - m0)
    probs = e / jnp.sum(e)
    probs = jnp.where(probs >= _TAU, probs, 0.0)

    cnt = jnp.sum(jnp.where(probs >= _TAU, 1, 0))
    cnt = jnp.minimum(cnt, _TOPK)

    probs_p = jnp.concatenate([probs, jnp.full((1, 24), -1.0, jnp.float32)], axis=1)
    P8 = jnp.concatenate([probs_p[:, i * 128:(i + 1) * 128] for i in range(8)], axis=0)
    fi = (lax.broadcasted_iota(jnp.int32, (8, 128), 0) * 128
          + lax.broadcasted_iota(jnp.int32, (8, 128), 1))
    l64 = lax.broadcasted_iota(jnp.int32, (1, 64), 1)
    col_t = lax.broadcasted_iota(jnp.int32, (64, 1), 0)

    def body(t, carry):
        P, wcol, irow, den = carry
        m = jnp.max(P, axis=(0, 1), keepdims=True)
        am = jnp.min(jnp.where(P == m, fi, jnp.int32(1 << 30)),
                     axis=(0, 1), keepdims=True)
        den = den + m
        wcol = wcol + jnp.where(col_t == t, m, 0.0)
        irow = irow + jnp.where(l64 == t, am, 0)
        P = jnp.where(fi == am, -1.0, P)
        return P, wcol, irow, den

    init = (P8,
            jnp.zeros((64, 1), jnp.float32),
            jnp.zeros((1, 64), jnp.int32),
            jnp.zeros((1, 1), jnp.float32))
    _, wcol, irow, den = lax.fori_loop(0, cnt, body, init)

    h_row = lax.broadcasted_iota(jnp.int32, (_KPAD * _R, 64), 0) // _R
    h_col = lax.broadcasted_iota(jnp.int32, (_KPAD * _R, 64), 1)
    H = jnp.where(h_row == h_col, 1.0, 0.0)
    wsc = wcol / (den + 1e-9) * _SCALING
    w_ref[...] = lax.dot_general(H, wsc, (((1,), (0,)), ((), ())),
                                 preferred_element_type=jnp.float32)

    i_row = lax.broadcasted_iota(jnp.int32, (8, 64), 0)
    i_col = lax.broadcasted_iota(jnp.int32, (8, 64), 1)
    IR = jnp.where(i_col // 8 == i_row, 1.0, 0.0) * irow.astype(jnp.float32)
    s_row = lax.broadcasted_iota(jnp.int32, (64, 128), 0)
    s_col = lax.broadcasted_iota(jnp.int32, (64, 128), 1)
    S = jnp.where(s_row % 8 == s_col // _R, 1.0, 0.0)
    irep8 = lax.dot_general(IR, S, (((1,), (0,)), ((), ())),
                            preferred_element_type=jnp.float32)
    lane_r = lax.broadcasted_iota(jnp.int32, (8, 128), 1) % _R
    ria_ref[...] = irep8.astype(jnp.int32) * _R + lane_r
    cnt_ref[...] = jnp.zeros((1, 128), jnp.int32) + cnt
    idx_ref[...] = jnp.concatenate([irow, jnp.zeros((1, 64), jnp.int32)], axis=1)


_routing = pl.pallas_call(
    _routing_body,
    out_shape=[
        jax.ShapeDtypeStruct((_KPAD * _R, 1), jnp.float32),
        jax.ShapeDtypeStruct((8, 128), jnp.int32),
        jax.ShapeDtypeStruct((1, 128), jnp.int32),
        jax.ShapeDtypeStruct((1, 128), jnp.int32),
    ],
)


_ROWS_PER_W = (_KPAD * _R) // 16
_RCHUNK = 32


@functools.cache
def _make_sc_gather():
    return pl.kernel(
        _sc_gather_body,
        mesh=plsc.VectorSubcoreMesh(core_axis_name="c", subcore_axis_name="s",
                                    num_cores=1),
        out_type=[
            jax.ShapeDtypeStruct((_KPAD * _R, _D), jnp.float32),
            jax.ShapeDtypeStruct((_KPAD * _R, _D), jnp.float32),
        ],
        scratch_types=[
            pltpu.VMEM((16,), jnp.int32),
            pltpu.VMEM((_RCHUNK,), jnp.int32),
            pltpu.VMEM((_RCHUNK, _D), jnp.float32),
            pltpu.VMEM((_RCHUNK, _D), jnp.float32),
            pltpu.SemaphoreType.DMA,
            pltpu.SemaphoreType.DMA,
        ],
    )


def _sc_gather_body(a2d, b2d, ria_hbm, cnt_hbm, a_out, b_out,
                    cv, ria_v, abuf, bbuf, asem, bsem):
    wid = lax.axis_index("s")
    pltpu.sync_copy(cnt_hbm.at[pl.ds(0, 16)], cv)
    cnt = cv[...][0]

    for u in range(_ROWS_PER_W // _RCHUNK):
        @pl.when((wid * 4 + u * 2) * _R < cnt * _R)
        def _():
            sl = pl.ds(wid * _ROWS_PER_W + u * _RCHUNK, _RCHUNK)
            pltpu.sync_copy(ria_hbm.at[sl], ria_v)
            a_copy = pltpu.make_async_copy(a2d.at[ria_v], abuf, asem)
            b_copy = pltpu.make_async_copy(b2d.at[ria_v], bbuf, bsem)
            a_copy.start()
            b_copy.start()
            a_copy.wait()
            pltpu.sync_copy(abuf, a_out.at[sl])
            b_copy.wait()
            pltpu.sync_copy(bbuf, b_out.at[sl])


_MBLK = 256


def _merge_body(ksize, b_ref, a_ref, w_ref, cnt_ref, base_ref, o_ref):
    cnt = cnt_ref[0, 0]
    krow = lax.broadcasted_iota(jnp.int32, (ksize, 1), 0)
    live = krow < cnt * _R
    scaled = jnp.where(live, b_ref[...] * w_ref[...], 0.0)
    amask = jnp.where(live, a_ref[...], 0.0)
    dn = (((0,), (0,)), ((), ()))
    o_ref[...] = base_ref[...] + lax.dot_general(
        scaled, amask, dn, preferred_element_type=jnp.float32)


def _make_merge(ksize):
    return pl.pallas_call(
        functools.partial(_merge_body, ksize),
        grid=(_D // _MBLK,),
        in_specs=[
            pl.BlockSpec((ksize, _MBLK), lambda i: (0, i)),
            pl.BlockSpec((ksize, _D), lambda i: (0, 0)),
            pl.BlockSpec((ksize, 1), lambda i: (0, 0)),
            pl.BlockSpec((1, 128), lambda i: (0, 0)),
            pl.BlockSpec((_MBLK, _D), lambda i: (i, 0)),
        ],
        out_specs=pl.BlockSpec((_MBLK, _D), lambda i: (i, 0)),
        out_shape=jax.ShapeDtypeStruct((_D, _D), jnp.float32),
    )


def _small_body(idx_ref, cnt_ref, a2d_ref, b2d_ref, w_ref, base_ref, o_ref,
                ag_s, bg_s, sem):
    @pl.when(pl.program_id(0) == 0)
    def _():
        copies = []
        for s in range(_KSMALL):
            cid = idx_ref[0, s]
            copies.append(pltpu.make_async_copy(
                a2d_ref.at[pl.ds(cid * _R, _R)], ag_s.at[pl.ds(s * _R, _R)], sem))
            copies.append(pltpu.make_async_copy(
                b2d_ref.at[pl.ds(cid * _R, _R)], bg_s.at[pl.ds(s * _R, _R)], sem))
        for cp in copies:
            cp.start()
        for cp in copies:
            cp.wait()

    i = pl.program_id(0)
    cnt = cnt_ref[0, 0]
    krow = lax.broadcasted_iota(jnp.int32, (_KSMALL * _R, 1), 0)
    live = krow < cnt * _R
    bg_blk = bg_s[:, pl.ds(i * _MBLK, _MBLK)]
    scaled = jnp.where(live, bg_blk * w_ref[...], 0.0)
    amask = jnp.where(live, ag_s[...], 0.0)
    dn = (((0,), (0,)), ((), ()))
    o_ref[...] = base_ref[...] + lax.dot_general(
        scaled, amask, dn, preferred_element_type=jnp.float32)


_small_merge = pl.pallas_call(
    _small_body,
    grid=(_D // _MBLK,),
    in_specs=[
        pl.BlockSpec(memory_space=pltpu.SMEM),
        pl.BlockSpec(memory_space=pltpu.SMEM),
        pl.BlockSpec(memory_space=pl.ANY),
        pl.BlockSpec(memory_space=pl.ANY),
        pl.BlockSpec((_KSMALL * _R, 1), lambda i: (0, 0)),
        pl.BlockSpec((_MBLK, _D), lambda i: (i, 0)),
    ],
    out_specs=pl.BlockSpec((_MBLK, _D), lambda i: (i, 0)),
    out_shape=jax.ShapeDtypeStruct((_D, _D), jnp.float32),
    scratch_shapes=[
        pltpu.VMEM((_KSMALL * _R, _D), jnp.float32),
        pltpu.VMEM((_KSMALL * _R, _D), jnp.float32),
        pltpu.SemaphoreType.DMA,
    ],
)


def kernel(q, corpus, A_all, B_all, W_base):
    wrow, ria8, cnt, idxs = _routing(q, corpus)
    a2d = A_all.reshape(_N * _R, _D)
    b2d = jnp.swapaxes(B_all, 1, 2).reshape(_N * _R, _D)

    def small_path():
        return _small_merge(idxs, cnt, a2d, b2d, wrow, W_base)

    def full_path():
        ag, bg = _make_sc_gather()(a2d, b2d, ria8.reshape(-1), cnt.reshape(-1))
        return _make_merge(_KPAD * _R)(bg, ag, wrow, cnt, W_base)

    return lax.cond(cnt[0, 0] <= _KSMALL, small_path, full_path)


# speedup vs baseline: 1.0256x; 1.0256x over previous
import functools

import jax
import jax.numpy as jnp
from jax import lax
from jax.experimental import pallas as pl
from jax.experimental.pallas import tpu as pltpu
from jax.experimental.pallas import tpu_sc as plsc

_N = 1000
_D = 1024
_R = 16
_TOPK = 50
_KPAD = 64
_BETA2 = 0.04
_TAU = 0.01
_SCALING = 2.0
_KSMALL = 4


def _routing_body(q_ref, c_ref, w_ref, ria_ref, cnt_ref, idx_ref):
    q = q_ref[...]
    C = c_ref[...]
    qnorm = jnp.sqrt(jnp.sum(q * q)) + 1e-9
    ones = jnp.ones((1, _D), jnp.float32)
    dn = (((1,), (1,)), ((), ()))
    dots = lax.dot_general(q, C, dn, preferred_element_type=jnp.float32)
    cn2 = lax.dot_general(ones, C * C, dn, preferred_element_type=jnp.float32)
    cnorm = jnp.sqrt(cn2) + 1e-9
    sim = dots / (qnorm * cnorm * _BETA2)
    m0 = jnp.max(sim)
    e = jnp.exp(sim - m0)
    probs = e / jnp.sum(e)
    probs = jnp.where(probs >= _TAU, probs, 0.0)

    cnt = jnp.sum(jnp.where(probs >= _TAU, 1, 0))
    cnt = jnp.minimum(cnt, _TOPK)

    probs_p = jnp.concatenate([probs, jnp.full((1, 24), -1.0, jnp.float32)], axis=1)
    P8 = jnp.concatenate([probs_p[:, i * 128:(i + 1) * 128] for i in range(8)], axis=0)
    fi = (lax.broadcasted_iota(jnp.int32, (8, 128), 0) * 128
          + lax.broadcasted_iota(jnp.int32, (8, 128), 1))
    l64 = lax.broadcasted_iota(jnp.int32, (1, 64), 1)
    col_t = lax.broadcasted_iota(jnp.int32, (64, 1), 0)

    def body(t, carry):
        P, wcol, irow, den = carry
        m = jnp.max(P, axis=(0, 1), keepdims=True)
        am = jnp.min(jnp.where(P == m, fi, jnp.int32(1 << 30)),
                     axis=(0, 1), keepdims=True)
        den = den + m
        wcol = wcol + jnp.where(col_t == t, m, 0.0)
        irow = irow + jnp.where(l64 == t, am, 0)
        P = jnp.where(fi == am, -1.0, P)
        return P, wcol, irow, den

    init = (P8,
            jnp.zeros((64, 1), jnp.float32),
            jnp.zeros((1, 64), jnp.int32),
            jnp.zeros((1, 1), jnp.float32))
    _, wcol, irow, den = lax.fori_loop(0, cnt, body, init)

    h_row = lax.broadcasted_iota(jnp.int32, (_KPAD * _R, 64), 0) // _R
    h_col = lax.broadcasted_iota(jnp.int32, (_KPAD * _R, 64), 1)
    H = jnp.where(h_row == h_col, 1.0, 0.0)
    wsc = wcol / (den + 1e-9) * _SCALING
    w_ref[...] = lax.dot_general(H, wsc, (((1,), (0,)), ((), ())),
                                 preferred_element_type=jnp.float32)

    i_row = lax.broadcasted_iota(jnp.int32, (8, 64), 0)
    i_col = lax.broadcasted_iota(jnp.int32, (8, 64), 1)
    IR = jnp.where(i_col // 8 == i_row, 1.0, 0.0) * irow.astype(jnp.float32)
    s_row = lax.broadcasted_iota(jnp.int32, (64, 128), 0)
    s_col = lax.broadcasted_iota(jnp.int32, (64, 128), 1)
    S = jnp.where(s_row % 8 == s_col // _R, 1.0, 0.0)
    irep8 = lax.dot_general(IR, S, (((1,), (0,)), ((), ())),
                            preferred_element_type=jnp.float32)
    lane_r = lax.broadcasted_iota(jnp.int32, (8, 128), 1) % _R
    ria_ref[...] = irep8.astype(jnp.int32) * _R + lane_r
    cnt_ref[...] = jnp.zeros((1, 128), jnp.int32) + cnt
    idx_ref[...] = jnp.concatenate([irow, jnp.zeros((1, 64), jnp.int32)], axis=1)


_routing = pl.pallas_call(
    _routing_body,
    out_shape=[
        jax.ShapeDtypeStruct((_KPAD * _R, 1), jnp.float32),
        jax.ShapeDtypeStruct((8, 128), jnp.int32),
        jax.ShapeDtypeStruct((1, 128), jnp.int32),
        jax.ShapeDtypeStruct((1, 128), jnp.int32),
    ],
)


_ROWS_PER_W = (_KPAD * _R) // 16
_RCHUNK = 32


@functools.cache
def _make_sc_gather():
    return pl.kernel(
        _sc_gather_body,
        mesh=plsc.VectorSubcoreMesh(core_axis_name="c", subcore_axis_name="s",
                                    num_cores=1),
        out_type=[
            jax.ShapeDtypeStruct((_KPAD * _R, _D), jnp.float32),
            jax.ShapeDtypeStruct((_KPAD * _R, _D), jnp.float32),
        ],
        scratch_types=[
            pltpu.VMEM((16,), jnp.int32),
            pltpu.VMEM((_RCHUNK,), jnp.int32),
            pltpu.VMEM((_RCHUNK, _D), jnp.float32),
            pltpu.VMEM((_RCHUNK, _D), jnp.float32),
            pltpu.SemaphoreType.DMA,
            pltpu.SemaphoreType.DMA,
        ],
    )


def _sc_gather_body(a2d, b2d, ria_hbm, cnt_hbm, a_out, b_out,
                    cv, ria_v, abuf, bbuf, asem, bsem):
    wid = lax.axis_index("s")
    pltpu.sync_copy(cnt_hbm.at[pl.ds(0, 16)], cv)
    cnt = cv[...][0]

    for u in range(_ROWS_PER_W // _RCHUNK):
        @pl.when((wid * 4 + u * 2) * _R < cnt * _R)
        def _():
            sl = pl.ds(wid * _ROWS_PER_W + u * _RCHUNK, _RCHUNK)
            pltpu.sync_copy(ria_hbm.at[sl], ria_v)
            a_copy = pltpu.make_async_copy(a2d.at[ria_v], abuf, asem)
            b_copy = pltpu.make_async_copy(b2d.at[ria_v], bbuf, bsem)
            a_copy.start()
            b_copy.start()
            a_copy.wait()
            pltpu.sync_copy(abuf, a_out.at[sl])
            b_copy.wait()
            pltpu.sync_copy(bbuf, b_out.at[sl])


_MBLK = 1024


def _merge_body(ksize, b_ref, a_ref, w_ref, cnt_ref, base_ref, o_ref):
    cnt = cnt_ref[0, 0]
    krow = lax.broadcasted_iota(jnp.int32, (ksize, 1), 0)
    live = krow < cnt * _R
    scaled = jnp.where(live, b_ref[...] * w_ref[...], 0.0)
    amask = jnp.where(live, a_ref[...], 0.0)
    dn = (((0,), (0,)), ((), ()))
    o_ref[...] = base_ref[...] + lax.dot_general(
        scaled, amask, dn, preferred_element_type=jnp.float32)


def _make_merge(ksize):
    return pl.pallas_call(
        functools.partial(_merge_body, ksize),
        grid=(_D // _MBLK,),
        in_specs=[
            pl.BlockSpec((ksize, _MBLK), lambda i: (0, i)),
            pl.BlockSpec((ksize, _D), lambda i: (0, 0)),
            pl.BlockSpec((ksize, 1), lambda i: (0, 0)),
            pl.BlockSpec((1, 128), lambda i: (0, 0)),
            pl.BlockSpec((_MBLK, _D), lambda i: (i, 0)),
        ],
        out_specs=pl.BlockSpec((_MBLK, _D), lambda i: (i, 0)),
        out_shape=jax.ShapeDtypeStruct((_D, _D), jnp.float32),
    )


def _small_body(idx_ref, cnt_ref, a2d_ref, b2d_ref, w_ref, base_ref, o_ref,
                ag_s, bg_s, sem):
    @pl.when(pl.program_id(0) == 0)
    def _():
        copies = []
        for s in range(_KSMALL):
            cid = idx_ref[0, s]
            copies.append(pltpu.make_async_copy(
                a2d_ref.at[pl.ds(cid * _R, _R)], ag_s.at[pl.ds(s * _R, _R)], sem))
            copies.append(pltpu.make_async_copy(
                b2d_ref.at[pl.ds(cid * _R, _R)], bg_s.at[pl.ds(s * _R, _R)], sem))
        for cp in copies:
            cp.start()
        for cp in copies:
            cp.wait()

    i = pl.program_id(0)
    cnt = cnt_ref[0, 0]
    krow = lax.broadcasted_iota(jnp.int32, (_KSMALL * _R, 1), 0)
    live = krow < cnt * _R
    bg_blk = bg_s[:, pl.ds(i * _MBLK, _MBLK)]
    scaled = jnp.where(live, bg_blk * w_ref[...], 0.0)
    amask = jnp.where(live, ag_s[...], 0.0)
    dn = (((0,), (0,)), ((), ()))
    o_ref[...] = base_ref[...] + lax.dot_general(
        scaled, amask, dn, preferred_element_type=jnp.float32)


_small_merge = pl.pallas_call(
    _small_body,
    grid=(_D // _MBLK,),
    in_specs=[
        pl.BlockSpec(memory_space=pltpu.SMEM),
        pl.BlockSpec(memory_space=pltpu.SMEM),
        pl.BlockSpec(memory_space=pl.ANY),
        pl.BlockSpec(memory_space=pl.ANY),
        pl.BlockSpec((_KSMALL * _R, 1), lambda i: (0, 0)),
        pl.BlockSpec((_MBLK, _D), lambda i: (i, 0)),
    ],
    out_specs=pl.BlockSpec((_MBLK, _D), lambda i: (i, 0)),
    out_shape=jax.ShapeDtypeStruct((_D, _D), jnp.float32),
    scratch_shapes=[
        pltpu.VMEM((_KSMALL * _R, _D), jnp.float32),
        pltpu.VMEM((_KSMALL * _R, _D), jnp.float32),
        pltpu.SemaphoreType.DMA,
    ],
)


def kernel(q, corpus, A_all, B_all, W_base):
    wrow, ria8, cnt, idxs = _routing(q, corpus)
    a2d = A_all.reshape(_N * _R, _D)
    b2d = jnp.swapaxes(B_all, 1, 2).reshape(_N * _R, _D)

    def small_path():
        return _small_merge(idxs, cnt, a2d, b2d, wrow, W_base)

    def full_path():
        ag, bg = _make_sc_gather()(a2d, b2d, ria8.reshape(-1), cnt.reshape(-1))
        return _make_merge(_KPAD * _R)(bg, ag, wrow, cnt, W_base)

    return lax.cond(cnt[0, 0] <= _KSMALL, small_path, full_path)


# speedup vs baseline: 1.0632x; 1.0366x over previous
import functools

import jax
import jax.numpy as jnp
from jax import lax
from jax.experimental import pallas as pl
from jax.experimental.pallas import tpu as pltpu
from jax.experimental.pallas import tpu_sc as plsc

_N = 1000
_D = 1024
_R = 16
_TOPK = 50
_KPAD = 64
_BETA2 = 0.04
_TAU = 0.01
_SCALING = 2.0
_KSMALL = 4


def _routing_body(q_ref, c_ref, w_ref, ria_ref, cnt_ref, idx_ref):
    q = q_ref[...]
    C = c_ref[...]
    qnorm = jnp.sqrt(jnp.sum(q * q)) + 1e-9
    ones = jnp.ones((1, _D), jnp.float32)
    dn = (((1,), (1,)), ((), ()))
    dots = lax.dot_general(q, C, dn, preferred_element_type=jnp.float32)
    cn2 = lax.dot_general(ones, C * C, dn, preferred_element_type=jnp.float32)
    cnorm = jnp.sqrt(cn2) + 1e-9
    sim = dots / (qnorm * cnorm * _BETA2)
    m0 = jnp.max(sim)
    e = jnp.exp(sim - m0)
    probs = e / jnp.sum(e)
    probs = jnp.where(probs >= _TAU, probs, 0.0)

    cnt = jnp.sum(jnp.where(probs >= _TAU, 1, 0))
    cnt = jnp.minimum(cnt, _TOPK)

    probs_p = jnp.concatenate([probs, jnp.full((1, 24), -1.0, jnp.float32)], axis=1)
    P8 = jnp.concatenate([probs_p[:, i * 128:(i + 1) * 128] for i in range(8)], axis=0)
    fi = (lax.broadcasted_iota(jnp.int32, (8, 128), 0) * 128
          + lax.broadcasted_iota(jnp.int32, (8, 128), 1))
    l64 = lax.broadcasted_iota(jnp.int32, (1, 64), 1)
    col_t = lax.broadcasted_iota(jnp.int32, (64, 1), 0)

    def body(t, carry):
        P, wcol, irow, den = carry
        m = jnp.max(P, axis=(0, 1), keepdims=True)
        am = jnp.min(jnp.where(P == m, fi, jnp.int32(1 << 30)),
                     axis=(0, 1), keepdims=True)
        den = den + m
        wcol = wcol + jnp.where(col_t == t, m, 0.0)
        irow = irow + jnp.where(l64 == t, am, 0)
        P = jnp.where(fi == am, -1.0, P)
        return P, wcol, irow, den

    init = (P8,
            jnp.zeros((64, 1), jnp.float32),
            jnp.zeros((1, 64), jnp.int32),
            jnp.zeros((1, 1), jnp.float32))
    _, wcol, irow, den = lax.fori_loop(0, cnt, body, init)

    h_row = lax.broadcasted_iota(jnp.int32, (_KPAD * _R, 64), 0) // _R
    h_col = lax.broadcasted_iota(jnp.int32, (_KPAD * _R, 64), 1)
    H = jnp.where(h_row == h_col, 1.0, 0.0)
    wsc = wcol / (den + 1e-9) * _SCALING
    w_ref[...] = lax.dot_general(H, wsc, (((1,), (0,)), ((), ())),
                                 preferred_element_type=jnp.float32)

    i_row = lax.broadcasted_iota(jnp.int32, (8, 64), 0)
    i_col = lax.broadcasted_iota(jnp.int32, (8, 64), 1)
    IR = jnp.where(i_col // 8 == i_row, 1.0, 0.0) * irow.astype(jnp.float32)
    s_row = lax.broadcasted_iota(jnp.int32, (64, 128), 0)
    s_col = lax.broadcasted_iota(jnp.int32, (64, 128), 1)
    S = jnp.where(s_row % 8 == s_col // _R, 1.0, 0.0)
    irep8 = lax.dot_general(IR, S, (((1,), (0,)), ((), ())),
                            preferred_element_type=jnp.float32)
    lane_r = lax.broadcasted_iota(jnp.int32, (8, 128), 1) % _R
    ria_ref[...] = irep8.astype(jnp.int32) * _R + lane_r
    cnt_ref[...] = jnp.zeros((1, 128), jnp.int32) + cnt
    idx_ref[...] = jnp.concatenate([irow, jnp.zeros((1, 64), jnp.int32)], axis=1)


_routing = pl.pallas_call(
    _routing_body,
    out_shape=[
        jax.ShapeDtypeStruct((_KPAD * _R, 1), jnp.float32),
        jax.ShapeDtypeStruct((8, 128), jnp.int32),
        jax.ShapeDtypeStruct((1, 128), jnp.int32),
        jax.ShapeDtypeStruct((1, 128), jnp.int32),
    ],
)


_ROWS_PER_W = (_KPAD * _R) // 16
_RCHUNK = 32


@functools.cache
def _make_sc_gather():
    return pl.kernel(
        _sc_gather_body,
        mesh=plsc.VectorSubcoreMesh(core_axis_name="c", subcore_axis_name="s",
                                    num_cores=1),
        out_type=[
            jax.ShapeDtypeStruct((_KPAD * _R, _D), jnp.float32),
            jax.ShapeDtypeStruct((_KPAD * _R, _D), jnp.float32),
        ],
        scratch_types=[
            pltpu.VMEM((16,), jnp.int32),
            pltpu.VMEM((_RCHUNK,), jnp.int32),
            pltpu.VMEM((_RCHUNK, _D), jnp.float32),
            pltpu.VMEM((_RCHUNK, _D), jnp.float32),
            pltpu.SemaphoreType.DMA,
            pltpu.SemaphoreType.DMA,
        ],
    )


def _sc_gather_body(a2d, b2d, ria_hbm, cnt_hbm, a_out, b_out,
                    cv, ria_v, abuf, bbuf, asem, bsem):
    wid = lax.axis_index("s")
    pltpu.sync_copy(cnt_hbm.at[pl.ds(0, 16)], cv)
    cnt = cv[...][0]

    for u in range(_ROWS_PER_W // _RCHUNK):
        @pl.when((wid * 4 + u * 2) * _R < cnt * _R)
        def _():
            sl = pl.ds(wid * _ROWS_PER_W + u * _RCHUNK, _RCHUNK)
            pltpu.sync_copy(ria_hbm.at[sl], ria_v)
            a_copy = pltpu.make_async_copy(a2d.at[ria_v], abuf, asem)
            b_copy = pltpu.make_async_copy(b2d.at[ria_v], bbuf, bsem)
            a_copy.start()
            b_copy.start()
            a_copy.wait()
            pltpu.sync_copy(abuf, a_out.at[sl])
            b_copy.wait()
            pltpu.sync_copy(bbuf, b_out.at[sl])


_MBLK = 512


def _merge_body(ksize, b_ref, a_ref, w_ref, cnt_ref, base_ref, o_ref):
    cnt = cnt_ref[0, 0]
    krow = lax.broadcasted_iota(jnp.int32, (ksize, 1), 0)
    live = krow < cnt * _R
    scaled = jnp.where(live, b_ref[...] * w_ref[...], 0.0)
    amask = jnp.where(live, a_ref[...], 0.0)
    dn = (((0,), (0,)), ((), ()))
    o_ref[...] = base_ref[...] + lax.dot_general(
        scaled, amask, dn, preferred_element_type=jnp.float32)


def _make_merge(ksize):
    return pl.pallas_call(
        functools.partial(_merge_body, ksize),
        grid=(_D // _MBLK,),
        in_specs=[
            pl.BlockSpec((ksize, _MBLK), lambda i: (0, i)),
            pl.BlockSpec((ksize, _D), lambda i: (0, 0)),
            pl.BlockSpec((ksize, 1), lambda i: (0, 0)),
            pl.BlockSpec((1, 128), lambda i: (0, 0)),
            pl.BlockSpec((_MBLK, _D), lambda i: (i, 0)),
        ],
        out_specs=pl.BlockSpec((_MBLK, _D), lambda i: (i, 0)),
        out_shape=jax.ShapeDtypeStruct((_D, _D), jnp.float32),
    )


def _small_body(idx_ref, cnt_ref, a2d_ref, b2d_ref, w_ref, base_ref, o_ref,
                ag_s, bg_s, sem):
    @pl.when(pl.program_id(0) == 0)
    def _():
        copies = []
        for s in range(_KSMALL):
            cid = idx_ref[0, s]
            copies.append(pltpu.make_async_copy(
                a2d_ref.at[pl.ds(cid * _R, _R)], ag_s.at[pl.ds(s * _R, _R)], sem))
            copies.append(pltpu.make_async_copy(
                b2d_ref.at[pl.ds(cid * _R, _R)], bg_s.at[pl.ds(s * _R, _R)], sem))
        for cp in copies:
            cp.start()
        for cp in copies:
            cp.wait()

    i = pl.program_id(0)
    cnt = cnt_ref[0, 0]
    krow = lax.broadcasted_iota(jnp.int32, (_KSMALL * _R, 1), 0)
    live = krow < cnt * _R
    bg_blk = bg_s[:, pl.ds(i * _MBLK, _MBLK)]
    scaled = jnp.where(live, bg_blk * w_ref[...], 0.0)
    amask = jnp.where(live, ag_s[...], 0.0)
    dn = (((0,), (0,)), ((), ()))
    o_ref[...] = base_ref[...] + lax.dot_general(
        scaled, amask, dn, preferred_element_type=jnp.float32)


_small_merge = pl.pallas_call(
    _small_body,
    grid=(_D // _MBLK,),
    in_specs=[
        pl.BlockSpec(memory_space=pltpu.SMEM),
        pl.BlockSpec(memory_space=pltpu.SMEM),
        pl.BlockSpec(memory_space=pl.ANY),
        pl.BlockSpec(memory_space=pl.ANY),
        pl.BlockSpec((_KSMALL * _R, 1), lambda i: (0, 0)),
        pl.BlockSpec((_MBLK, _D), lambda i: (i, 0)),
    ],
    out_specs=pl.BlockSpec((_MBLK, _D), lambda i: (i, 0)),
    out_shape=jax.ShapeDtypeStruct((_D, _D), jnp.float32),
    scratch_shapes=[
        pltpu.VMEM((_KSMALL * _R, _D), jnp.float32),
        pltpu.VMEM((_KSMALL * _R, _D), jnp.float32),
        pltpu.SemaphoreType.DMA,
    ],
)


def kernel(q, corpus, A_all, B_all, W_base):
    wrow, ria8, cnt, idxs = _routing(q, corpus)
    a2d = A_all.reshape(_N * _R, _D)
    b2d = jnp.swapaxes(B_all, 1, 2).reshape(_N * _R, _D)

    def small_path():
        return _small_merge(idxs, cnt, a2d, b2d, wrow, W_base)

    def full_path():
        ag, bg = _make_sc_gather()(a2d, b2d, ria8.reshape(-1), cnt.reshape(-1))
        return _make_merge(_KPAD * _R)(bg, ag, wrow, cnt, W_base)

    return lax.cond(cnt[0, 0] <= _KSMALL, small_path, full_path)


# speedup vs baseline: 1.1016x; 1.0361x over previous
import functools

import jax
import jax.numpy as jnp
from jax import lax
from jax.experimental import pallas as pl
from jax.experimental.pallas import tpu as pltpu
from jax.experimental.pallas import tpu_sc as plsc

_N = 1000
_D = 1024
_R = 16
_TOPK = 50
_KPAD = 64
_BETA2 = 0.04
_TAU = 0.01
_SCALING = 2.0
_KSMALL = 4


def _routing_math(q, C):
    qnorm = jnp.sqrt(jnp.sum(q * q)) + 1e-9
    ones = jnp.ones((1, _D), jnp.float32)
    dn = (((1,), (1,)), ((), ()))
    dots = lax.dot_general(q, C, dn, preferred_element_type=jnp.float32)
    cn2 = lax.dot_general(ones, C * C, dn, preferred_element_type=jnp.float32)
    cnorm = jnp.sqrt(cn2) + 1e-9
    sim = dots / (qnorm * cnorm * _BETA2)
    m0 = jnp.max(sim)
    e = jnp.exp(sim - m0)
    probs = e / jnp.sum(e)
    probs = jnp.where(probs >= _TAU, probs, 0.0)

    cnt = jnp.minimum(jnp.sum(jnp.where(probs >= _TAU, 1, 0)), _TOPK)

    probs_p = jnp.concatenate([probs, jnp.full((1, 24), -1.0, jnp.float32)], axis=1)
    P8 = jnp.concatenate([probs_p[:, i * 128:(i + 1) * 128] for i in range(8)], axis=0)
    fi = (lax.broadcasted_iota(jnp.int32, (8, 128), 0) * 128
          + lax.broadcasted_iota(jnp.int32, (8, 128), 1))
    l64 = lax.broadcasted_iota(jnp.int32, (1, 64), 1)
    col_t = lax.broadcasted_iota(jnp.int32, (64, 1), 0)

    def body(t, carry):
        P, wcol, irow, den = carry
        m = jnp.max(P, axis=(0, 1), keepdims=True)
        am = jnp.min(jnp.where(P == m, fi, jnp.int32(1 << 30)),
                     axis=(0, 1), keepdims=True)
        den = den + m
        wcol = wcol + jnp.where(col_t == t, m, 0.0)
        irow = irow + jnp.where(l64 == t, am, 0)
        P = jnp.where(fi == am, -1.0, P)
        return P, wcol, irow, den

    init = (P8,
            jnp.zeros((64, 1), jnp.float32),
            jnp.zeros((1, 64), jnp.int32),
            jnp.zeros((1, 1), jnp.float32))
    _, wcol, irow, den = lax.fori_loop(0, cnt, body, init)

    h_row = lax.broadcasted_iota(jnp.int32, (_KPAD * _R, 64), 0) // _R
    h_col = lax.broadcasted_iota(jnp.int32, (_KPAD * _R, 64), 1)
    H = jnp.where(h_row == h_col, 1.0, 0.0)
    wsc = wcol / (den + 1e-9) * _SCALING
    wfull = lax.dot_general(H, wsc, (((1,), (0,)), ((), ())),
                            preferred_element_type=jnp.float32)

    i_row = lax.broadcasted_iota(jnp.int32, (8, 64), 0)
    i_col = lax.broadcasted_iota(jnp.int32, (8, 64), 1)
    IR = jnp.where(i_col // 8 == i_row, 1.0, 0.0) * irow.astype(jnp.float32)
    s_row = lax.broadcasted_iota(jnp.int32, (64, 128), 0)
    s_col = lax.broadcasted_iota(jnp.int32, (64, 128), 1)
    S = jnp.where(s_row % 8 == s_col // _R, 1.0, 0.0)
    irep8 = lax.dot_general(IR, S, (((1,), (0,)), ((), ())),
                            preferred_element_type=jnp.float32)
    lane_r = lax.broadcasted_iota(jnp.int32, (8, 128), 1) % _R
    ria8 = irep8.astype(jnp.int32) * _R + lane_r
    return wfull, irow, cnt, ria8


def _fused_body(q_ref, c_ref, a2d_ref, b2d_ref, base_ref,
                o_ref, w_ref, ria_ref, cnt_ref,
                ag_s, bg_s, sem):
    wfull, irow, cnt, ria8 = _routing_math(q_ref[...], c_ref[...])
    w_ref[...] = wfull
    ria_ref[...] = ria8
    cnt_ref[...] = jnp.zeros((1, 128), jnp.int32) + cnt

    copies = []
    for s in range(_KSMALL):
        cid = irow[0, s]
        copies.append(pltpu.make_async_copy(
            a2d_ref.at[pl.ds(cid * _R, _R)], ag_s.at[pl.ds(s * _R, _R)], sem))
        copies.append(pltpu.make_async_copy(
            b2d_ref.at[pl.ds(cid * _R, _R)], bg_s.at[pl.ds(s * _R, _R)], sem))
    for cp in copies:
        cp.start()
    for cp in copies:
        cp.wait()

    krow = lax.broadcasted_iota(jnp.int32, (_KSMALL * _R, 1), 0)
    live = krow < cnt * _R
    wsmall = wfull[0:_KSMALL * _R, :]
    scaled = jnp.where(live, bg_s[...] * wsmall, 0.0)
    amask = jnp.where(live, ag_s[...], 0.0)
    dn = (((0,), (0,)), ((), ()))
    o_ref[...] = base_ref[...] + lax.dot_general(
        scaled, amask, dn, preferred_element_type=jnp.float32)


_fused = pl.pallas_call(
    _fused_body,
    in_specs=[
        pl.BlockSpec((1, _D), lambda: (0, 0)),
        pl.BlockSpec((_N, _D), lambda: (0, 0)),
        pl.BlockSpec(memory_space=pl.ANY),
        pl.BlockSpec(memory_space=pl.ANY),
        pl.BlockSpec((_D, _D), lambda: (0, 0)),
    ],
    out_specs=[
        pl.BlockSpec((_D, _D), lambda: (0, 0)),
        pl.BlockSpec((_KPAD * _R, 1), lambda: (0, 0)),
        pl.BlockSpec((8, 128), lambda: (0, 0)),
        pl.BlockSpec((1, 128), lambda: (0, 0)),
    ],
    out_shape=[
        jax.ShapeDtypeStruct((_D, _D), jnp.float32),
        jax.ShapeDtypeStruct((_KPAD * _R, 1), jnp.float32),
        jax.ShapeDtypeStruct((8, 128), jnp.int32),
        jax.ShapeDtypeStruct((1, 128), jnp.int32),
    ],
    scratch_shapes=[
        pltpu.VMEM((_KSMALL * _R, _D), jnp.float32),
        pltpu.VMEM((_KSMALL * _R, _D), jnp.float32),
        pltpu.SemaphoreType.DMA,
    ],
)


_ROWS_PER_W = (_KPAD * _R) // 16
_RCHUNK = 32


@functools.cache
def _make_sc_gather():
    return pl.kernel(
        _sc_gather_body,
        mesh=plsc.VectorSubcoreMesh(core_axis_name="c", subcore_axis_name="s",
                                    num_cores=1),
        out_type=[
            jax.ShapeDtypeStruct((_KPAD * _R, _D), jnp.float32),
            jax.ShapeDtypeStruct((_KPAD * _R, _D), jnp.float32),
        ],
        scratch_types=[
            pltpu.VMEM((16,), jnp.int32),
            pltpu.VMEM((_RCHUNK,), jnp.int32),
            pltpu.VMEM((_RCHUNK, _D), jnp.float32),
            pltpu.VMEM((_RCHUNK, _D), jnp.float32),
            pltpu.SemaphoreType.DMA,
            pltpu.SemaphoreType.DMA,
        ],
    )


def _sc_gather_body(a2d, b2d, ria_hbm, cnt_hbm, a_out, b_out,
                    cv, ria_v, abuf, bbuf, asem, bsem):
    wid = lax.axis_index("s")
    pltpu.sync_copy(cnt_hbm.at[pl.ds(0, 16)], cv)
    cnt = cv[...][0]

    for u in range(_ROWS_PER_W // _RCHUNK):
        @pl.when((wid * 4 + u * 2) * _R < cnt * _R)
        def _():
            sl = pl.ds(wid * _ROWS_PER_W + u * _RCHUNK, _RCHUNK)
            pltpu.sync_copy(ria_hbm.at[sl], ria_v)
            a_copy = pltpu.make_async_copy(a2d.at[ria_v], abuf, asem)
            b_copy = pltpu.make_async_copy(b2d.at[ria_v], bbuf, bsem)
            a_copy.start()
            b_copy.start()
            a_copy.wait()
            pltpu.sync_copy(abuf, a_out.at[sl])
            b_copy.wait()
            pltpu.sync_copy(bbuf, b_out.at[sl])


_MBLK = 512


def _merge_body(ksize, b_ref, a_ref, w_ref, cnt_ref, base_ref, o_ref):
    cnt = cnt_ref[0, 0]
    krow = lax.broadcasted_iota(jnp.int32, (ksize, 1), 0)
    live = krow < cnt * _R
    scaled = jnp.where(live, b_ref[...] * w_ref[...], 0.0)
    amask = jnp.where(live, a_ref[...], 0.0)
    dn = (((0,), (0,)), ((), ()))
    o_ref[...] = base_ref[...] + lax.dot_general(
        scaled, amask, dn, preferred_element_type=jnp.float32)


def _make_merge(ksize):
    return pl.pallas_call(
        functools.partial(_merge_body, ksize),
        grid=(_D // _MBLK,),
        in_specs=[
            pl.BlockSpec((ksize, _MBLK), lambda i: (0, i)),
            pl.BlockSpec((ksize, _D), lambda i: (0, 0)),
            pl.BlockSpec((ksize, 1), lambda i: (0, 0)),
            pl.BlockSpec((1, 128), lambda i: (0, 0)),
            pl.BlockSpec((_MBLK, _D), lambda i: (i, 0)),
        ],
        out_specs=pl.BlockSpec((_MBLK, _D), lambda i: (i, 0)),
        out_shape=jax.ShapeDtypeStruct((_D, _D), jnp.float32),
    )


def kernel(q, corpus, A_all, B_all, W_base):
    a2d = A_all.reshape(_N * _R, _D)
    b2d = jnp.swapaxes(B_all, 1, 2).reshape(_N * _R, _D)
    out_small, wrow, ria8, cnt = _fused(q, corpus, a2d, b2d, W_base)

    def full_path():
        ag, bg = _make_sc_gather()(a2d, b2d, ria8.reshape(-1), cnt.reshape(-1))
        return _make_merge(_KPAD * _R)(bg, ag, wrow, cnt, W_base)

    return lax.cond(cnt[0, 0] <= _KSMALL, lambda: out_small, full_path)
